# Initial kernel scaffold; baseline (speedup 1.0000x reference)
import jax
import jax.numpy as jnp
from jax.experimental import pallas as pl


def kernel(x, edge_index, batch, params):
    raise NotImplementedError("write your pallas kernel here")


# speedup vs baseline: 2.0511x; 2.0511x over previous
import functools

import jax
import jax.numpy as jnp
from jax.experimental import pallas as pl
from jax.experimental.pallas import tpu as pltpu

_SELU_A = 1.6732632423543772
_SELU_S = 1.0507009873554805
_BN_S = 1.0 / (1.00001 ** 0.5)


def _mm_scale_body(h_ref, w_ref, dinv_ref, o_ref):
    hw = jnp.dot(h_ref[...], w_ref[...], preferred_element_type=jnp.float32)
    o_ref[...] = hw * dinv_ref[...]


def _mm_scale(h, W, dinv2d, block=1000):
    n, _ = h.shape
    o = W.shape[1]
    return pl.pallas_call(
        _mm_scale_body,
        grid=(n // block,),
        in_specs=[
            pl.BlockSpec((block, h.shape[1]), lambda i: (i, 0)),
            pl.BlockSpec(W.shape, lambda i: (0, 0)),
            pl.BlockSpec((block, 1), lambda i: (i, 0)),
        ],
        out_specs=pl.BlockSpec((block, o), lambda i: (i, 0)),
        out_shape=jax.ShapeDtypeStruct((n, o), jnp.float32),
    )(h, W, dinv2d)


def _post_gcn_body(part_ref, hws_ref, dinv_ref, b_ref, o_ref):
    z = dinv_ref[...] * (part_ref[...] + hws_ref[...]) + b_ref[...]
    neg = _SELU_A * (jnp.exp(jnp.minimum(z, 0.0)) - 1.0)
    o_ref[...] = _SELU_S * jnp.where(z > 0, z, neg)


def _post_gcn(partial, hws, dinv2d, b, block=1000):
    n, hdim = hws.shape
    return pl.pallas_call(
        _post_gcn_body,
        grid=(n // block,),
        in_specs=[
            pl.BlockSpec((block, hdim), lambda i: (i, 0)),
            pl.BlockSpec((block, hdim), lambda i: (i, 0)),
            pl.BlockSpec((block, 1), lambda i: (i, 0)),
            pl.BlockSpec((1, hdim), lambda i: (0, 0)),
        ],
        out_specs=pl.BlockSpec((block, hdim), lambda i: (i, 0)),
        out_shape=jax.ShapeDtypeStruct((n, hdim), jnp.float32),
    )(partial, hws, dinv2d, b.reshape(1, hdim))


def _readout_body(h_ref, w_ref, b_ref, r_ref, o_ref):
    z = jnp.dot(h_ref[...], w_ref[...], preferred_element_type=jnp.float32)
    z = z + b_ref[...]
    z = z - jnp.max(z, axis=-1, keepdims=True)
    e = jnp.exp(z)
    o_ref[...] = r_ref[...] + e / jnp.sum(e, axis=-1, keepdims=True)


def _readout_add(h, lW, lb, r, block=1000):
    n, hdim = h.shape
    p = lW.shape[1]
    return pl.pallas_call(
        _readout_body,
        grid=(n // block,),
        in_specs=[
            pl.BlockSpec((block, hdim), lambda i: (i, 0)),
            pl.BlockSpec(lW.shape, lambda i: (0, 0)),
            pl.BlockSpec((1, p), lambda i: (0, 0)),
            pl.BlockSpec((block, p), lambda i: (i, 0)),
        ],
        out_specs=pl.BlockSpec((block, p), lambda i: (i, 0)),
        out_shape=jax.ShapeDtypeStruct((n, p), jnp.float32),
    )(h, lW, lb.reshape(1, p), r)


def _cls_body(g_ref, w1, b1, g1, be1, w2, b2, g2, be2, w3, b3, g3, be3,
              w4, b4, o_ref):
    def lin(v, w, b):
        return jnp.dot(v, w[...], preferred_element_type=jnp.float32) + b[...]

    def bn(v, ga, be):
        return v * _BN_S * ga[...] + be[...]

    z = bn(jnp.maximum(lin(g_ref[...], w1, b1), 0.0), g1, be1)
    z = bn(jnp.maximum(lin(z, w2, b2), 0.0), g2, be2)
    z = bn(jnp.maximum(lin(z, w3, b3), 0.0), g3, be3)
    logits = lin(z, w4, b4)
    p = 1.0 / (1.0 + jnp.exp(-logits))
    p = p - jnp.max(p, axis=-1, keepdims=True)
    e = jnp.exp(p)
    o_ref[...] = e / jnp.sum(e, axis=-1, keepdims=True)


def _classifier(g, params):
    P = params
    args = [g]
    for i, names in enumerate((('cW1', 'cb1', 'g1', 'be1'),
                               ('cW2', 'cb2', 'g2', 'be2'),
                               ('cW3', 'cb3', 'g3', 'be3'))):
        w, b, ga, be = (P[k] for k in names)
        args += [w, b.reshape(1, -1), ga.reshape(1, -1), be.reshape(1, -1)]
    args += [P['cW4'], P['cb4'].reshape(1, -1)]
    nout = P['cW4'].shape[1]
    specs = [pl.BlockSpec(a.shape, lambda i: (0,) * a.ndim) for a in args]
    return pl.pallas_call(
        _cls_body,
        grid=(1,),
        in_specs=specs,
        out_specs=pl.BlockSpec((g.shape[0], nout), lambda i: (0, 0)),
        out_shape=jax.ShapeDtypeStruct((g.shape[0], nout), jnp.float32),
    )(*args)


def _seg_count(col, n):
    return jax.ops.segment_sum(jnp.ones(col.shape, jnp.float32), col,
                               num_segments=n)


def _seg_sum_rows(vals, row, col, n):
    return jax.ops.segment_sum(vals[row], col, num_segments=n)


def _seg_max_rows(vals, row, col, n):
    m = jax.ops.segment_max(vals[row], col, num_segments=n)
    return jnp.maximum(vals, m)


def _batch_sum(vals, batch, g):
    return jax.ops.segment_sum(vals, batch, num_segments=g)


def kernel(x, edge_index, batch, params):
    n = x.shape[0]
    g = 64
    row, col = edge_index[0], edge_index[1]

    deg = _seg_count(col, n) + 1.0
    dinv = jax.lax.rsqrt(deg).reshape(n, 1)

    h = x
    readout = None
    for i in range(1, 5):
        W, b = params[f'W{i}'], params[f'b{i}']
        lW, lb = params[f'lW{i}'], params[f'lb{i}']
        hws = _mm_scale(h, W, dinv)
        partial = _seg_sum_rows(hws, row, col, n)
        h_act = _post_gcn(partial, hws, dinv, b)
        h = _seg_max_rows(h_act, row, col, n)
        if readout is None:
            readout = jnp.zeros((n, lW.shape[1]), jnp.float32)
        readout = _readout_add(h, lW, lb, readout)

    gsum = _batch_sum(readout, batch, g)
    return _classifier(gsum, params)


# speedup vs baseline: 3.2695x; 1.5941x over previous
import functools

import jax
import jax.numpy as jnp
from jax import lax
from jax.experimental import pallas as pl
from jax.experimental.pallas import tpu as pltpu
from jax.experimental.pallas import tpu_sc as plsc

_NC, _NS = 2, 16
_NW = _NC * _NS
_SC_MESH = dict(mesh=plsc.VectorSubcoreMesh(core_axis_name="c",
                                            subcore_axis_name="s"))

_SELU_A = 1.6732632423543772
_SELU_S = 1.0507009873554805
_BN_S = 1.0 / (1.00001 ** 0.5)


def _mm_scale_body(h_ref, w_ref, dinv_ref, o_ref):
    hw = jnp.dot(h_ref[...], w_ref[...], preferred_element_type=jnp.float32)
    o_ref[...] = hw * dinv_ref[...]


def _mm_scale(h, W, dinv2d, block=1000):
    n, _ = h.shape
    o = W.shape[1]
    return pl.pallas_call(
        _mm_scale_body,
        grid=(n // block,),
        in_specs=[
            pl.BlockSpec((block, h.shape[1]), lambda i: (i, 0)),
            pl.BlockSpec(W.shape, lambda i: (0, 0)),
            pl.BlockSpec((block, 1), lambda i: (i, 0)),
        ],
        out_specs=pl.BlockSpec((block, o), lambda i: (i, 0)),
        out_shape=jax.ShapeDtypeStruct((n, o), jnp.float32),
    )(h, W, dinv2d)


def _dinv_body(p0_ref, p1_ref, o_ref):
    o_ref[...] = lax.rsqrt(1.0 + p0_ref[...] + p1_ref[...])


def _dinv_from_partials(p0, p1, block=1000):
    n = p0.shape[0]
    return pl.pallas_call(
        _dinv_body,
        grid=(n // block,),
        in_specs=[pl.BlockSpec((block, 1), lambda i: (i, 0))] * 2,
        out_specs=pl.BlockSpec((block, 1), lambda i: (i, 0)),
        out_shape=jax.ShapeDtypeStruct((n, 1), jnp.float32),
    )(p0, p1)


def _post_gcn_body(part_ref, hws_ref, dinv_ref, b_ref, o_ref):
    part = jnp.concatenate([part_ref[0], part_ref[1]], axis=-1)
    z = dinv_ref[...] * (part + hws_ref[...]) + b_ref[...]
    neg = _SELU_A * (jnp.exp(jnp.minimum(z, 0.0)) - 1.0)
    o_ref[...] = _SELU_S * jnp.where(z > 0, z, neg)


def _post_gcn(part3d, hws, dinv2d, b, block=1000):
    n, hdim = hws.shape
    half = hdim // 2
    return pl.pallas_call(
        _post_gcn_body,
        grid=(n // block,),
        in_specs=[
            pl.BlockSpec((2, block, half), lambda i: (0, i, 0)),
            pl.BlockSpec((block, hdim), lambda i: (i, 0)),
            pl.BlockSpec((block, 1), lambda i: (i, 0)),
            pl.BlockSpec((1, hdim), lambda i: (0, 0)),
        ],
        out_specs=pl.BlockSpec((block, hdim), lambda i: (i, 0)),
        out_shape=jax.ShapeDtypeStruct((n, hdim), jnp.float32),
    )(part3d, hws, dinv2d, b.reshape(1, hdim))


def _readout_body(h_ref, w_ref, b_ref, r_ref, o_ref):
    z = jnp.dot(h_ref[...], w_ref[...], preferred_element_type=jnp.float32)
    z = z + b_ref[...]
    z = z - jnp.max(z, axis=-1, keepdims=True)
    e = jnp.exp(z)
    o_ref[...] = r_ref[...] + e / jnp.sum(e, axis=-1, keepdims=True)


def _readout_add(h, lW, lb, r, block=1000):
    n, hdim = h.shape
    p = lW.shape[1]
    return pl.pallas_call(
        _readout_body,
        grid=(n // block,),
        in_specs=[
            pl.BlockSpec((block, hdim), lambda i: (i, 0)),
            pl.BlockSpec(lW.shape, lambda i: (0, 0)),
            pl.BlockSpec((1, p), lambda i: (0, 0)),
            pl.BlockSpec((block, p), lambda i: (i, 0)),
        ],
        out_specs=pl.BlockSpec((block, p), lambda i: (i, 0)),
        out_shape=jax.ShapeDtypeStruct((n, p), jnp.float32),
    )(h, lW, lb.reshape(1, p), r)


def _cls_body(ga_ref, gb_ref, w1, b1, g1, be1, w2, b2, g2, be2, w3, b3, g3, be3,
              w4, b4, o_ref):
    def lin(v, w, b):
        return jnp.dot(v, w[...], preferred_element_type=jnp.float32) + b[...]

    def bn(v, ga, be):
        return v * _BN_S * ga[...] + be[...]

    z = bn(jnp.maximum(lin(ga_ref[...] + gb_ref[...], w1, b1), 0.0), g1, be1)
    z = bn(jnp.maximum(lin(z, w2, b2), 0.0), g2, be2)
    z = bn(jnp.maximum(lin(z, w3, b3), 0.0), g3, be3)
    logits = lin(z, w4, b4)
    p = 1.0 / (1.0 + jnp.exp(-logits))
    p = p - jnp.max(p, axis=-1, keepdims=True)
    e = jnp.exp(p)
    o_ref[...] = e / jnp.sum(e, axis=-1, keepdims=True)


def _classifier(ga, gb, params):
    P = params
    args = [ga, gb]
    for names in (('cW1', 'cb1', 'g1', 'be1'),
                  ('cW2', 'cb2', 'g2', 'be2'),
                  ('cW3', 'cb3', 'g3', 'be3')):
        w, b, gm, be = (P[k] for k in names)
        args += [w, b.reshape(1, -1), gm.reshape(1, -1), be.reshape(1, -1)]
    args += [P['cW4'], P['cb4'].reshape(1, -1)]
    nout = P['cW4'].shape[1]
    specs = [pl.BlockSpec(a.shape, lambda i, _s=a.shape: (0,) * len(_s))
             for a in args]
    return pl.pallas_call(
        _cls_body,
        grid=(1,),
        in_specs=specs,
        out_specs=pl.BlockSpec((ga.shape[0], nout), lambda i: (0, 0)),
        out_shape=jax.ShapeDtypeStruct((ga.shape[0], nout), jnp.float32),
    )(*args)


def _zero_shared_rows(zeros_v, shared, sid, nrows_each, nrows_last):
    @pl.when(sid < _NS - 1)
    def _():
        pltpu.sync_copy(zeros_v, shared.at[pl.ds(sid * nrows_each,
                                                 nrows_each)])

    @pl.when(sid == _NS - 1)
    def _():
        pltpu.sync_copy(zeros_v.at[pl.ds(0, nrows_last)],
                        shared.at[pl.ds((_NS - 1) * nrows_each, nrows_last)])


def _sc_deg(col2d, zeros1d, ones1d, n):
    nchunk = col2d.shape[0]
    niter = (nchunk + _NW - 1) // _NW
    rows_each = ((n + _NS - 1) // _NS + 7) // 8 * 8
    rows_last = n - (_NS - 1) * rows_each

    @functools.partial(
        pl.kernel,
        out_type=jax.ShapeDtypeStruct((_NC * n,), jnp.float32),
        scratch_types=[
            pltpu.VMEM((128,), jnp.int32),
            pltpu.VMEM((128,), jnp.float32),
            pltpu.VMEM((rows_each,), jnp.float32),
            pltpu.VMEM_SHARED((n,), jnp.float32),
        ],
        **_SC_MESH,
    )
    def k(col_hbm, zeros_hbm, ones_hbm, out_hbm, idx_v, ones_v, zv, shared):
        cid = lax.axis_index("c")
        sid = lax.axis_index("s")
        wid = sid * _NC + cid
        pltpu.sync_copy(ones_hbm, ones_v)
        pltpu.sync_copy(zeros_hbm, zv)
        _zero_shared_rows(zv, shared, sid, rows_each, rows_last)
        plsc.subcore_barrier()

        def body(r, carry):
            ci = wid + _NW * r

            @pl.when(ci < nchunk)
            def _():
                pltpu.sync_copy(col_hbm.at[ci], idx_v)
                pltpu.sync_copy(ones_v, shared.at[idx_v], add=True)
            return carry

        lax.fori_loop(0, niter, body, 0)
        plsc.subcore_barrier()

        @pl.when(sid < _NS - 1)
        def _():
            pltpu.sync_copy(shared.at[pl.ds(sid * rows_each, rows_each)], zv)
            pltpu.sync_copy(
                zv, out_hbm.at[pl.ds(cid * n + sid * rows_each, rows_each)])

        @pl.when(sid == _NS - 1)
        def _():
            sl = pl.ds(0, rows_last)
            pltpu.sync_copy(
                shared.at[pl.ds((_NS - 1) * rows_each, rows_last)], zv.at[sl])
            pltpu.sync_copy(
                zv.at[sl],
                out_hbm.at[pl.ds(cid * n + (_NS - 1) * rows_each, rows_last)])

    return k(col2d, zeros1d, ones1d)


def _sc_scatter_rows(vals2h, row2d, col2d, zeros2d, n):
    nchunk = row2d.shape[0]
    niter = (nchunk + _NS - 1) // _NS
    half = vals2h.shape[1]
    rows_each = ((n + _NS - 1) // _NS + 7) // 8 * 8
    rows_last = n - (_NS - 1) * rows_each

    @functools.partial(
        pl.kernel,
        out_type=jax.ShapeDtypeStruct((_NC, n, half), jnp.float32),
        scratch_types=[
            pltpu.VMEM((128,), jnp.int32),
            pltpu.VMEM((128,), jnp.int32),
            pltpu.VMEM((128, half), jnp.float32),
            pltpu.VMEM((rows_each, half), jnp.float32),
            pltpu.VMEM_SHARED((n, half), jnp.float32),
            pltpu.SemaphoreType.DMA,
        ],
        compiler_params=pltpu.CompilerParams(use_tc_tiling_on_sc=False),
        **_SC_MESH,
    )
    def k(vals_hbm, row_hbm, col_hbm, zeros_hbm, out_hbm,
          idxr, idxc, rows_v, zv, shared, sem):
        cid = lax.axis_index("c")
        sid = lax.axis_index("s")
        pltpu.sync_copy(zeros_hbm, zv)
        _zero_shared_rows(zv, shared, sid, rows_each, rows_last)
        plsc.subcore_barrier()

        def body(r, carry):
            ci = sid + _NS * r

            @pl.when(ci < nchunk)
            def _():
                pltpu.sync_copy(row_hbm.at[ci], idxr)
                for q in range(8):
                    sl = pl.ds(q * 16, 16)
                    idxr[sl] = idxr[sl] * 2 + cid
                pltpu.async_copy(vals_hbm.at[idxr], rows_v, sem).wait()
                pltpu.sync_copy(col_hbm.at[ci], idxc)
                pltpu.sync_copy(rows_v, shared.at[idxc], add=True)
            return carry

        lax.fori_loop(0, niter, body, 0)
        plsc.subcore_barrier()

        @pl.when(sid < _NS - 1)
        def _():
            pltpu.sync_copy(shared.at[pl.ds(sid * rows_each, rows_each)], zv)
            pltpu.sync_copy(
                zv, out_hbm.at[cid, pl.ds(sid * rows_each, rows_each)])

        @pl.when(sid == _NS - 1)
        def _():
            sl = pl.ds(0, rows_last)
            pltpu.sync_copy(
                shared.at[pl.ds((_NS - 1) * rows_each, rows_last)], zv.at[sl])
            pltpu.sync_copy(
                zv.at[sl],
                out_hbm.at[cid, pl.ds((_NS - 1) * rows_each, rows_last)])

    return k(vals2h, row2d, col2d, zeros2d)


def _sc_batch_sum(vals, batch, zeros2d, g):
    n, p = vals.shape
    chunk = 128
    nchunk = n // chunk
    niter = (nchunk + _NW - 1) // _NW
    rows_each = g // _NS

    @functools.partial(
        pl.kernel,
        out_type=jax.ShapeDtypeStruct((_NC, g, p), jnp.float32),
        scratch_types=[
            pltpu.VMEM((chunk,), jnp.int32),
            pltpu.VMEM((chunk, p), jnp.float32),
            pltpu.VMEM((rows_each, p), jnp.float32),
            pltpu.VMEM_SHARED((g, p), jnp.float32),
        ],
        compiler_params=pltpu.CompilerParams(use_tc_tiling_on_sc=False),
        **_SC_MESH,
    )
    def k(vals_hbm, batch_hbm, zeros_hbm, out_hbm, idxb, val_v, zv, shared):
        cid = lax.axis_index("c")
        sid = lax.axis_index("s")
        wid = sid * _NC + cid
        pltpu.sync_copy(zeros_hbm, zv)
        pltpu.sync_copy(zv, shared.at[pl.ds(sid * rows_each, rows_each)])
        plsc.subcore_barrier()

        def body(r, carry):
            ci = wid + _NW * r

            @pl.when(ci < nchunk)
            def _():
                pltpu.sync_copy(batch_hbm.at[pl.ds(ci * chunk, chunk)], idxb)
                pltpu.sync_copy(vals_hbm.at[pl.ds(ci * chunk, chunk)], val_v)
                pltpu.sync_copy(val_v, shared.at[idxb], add=True)
            return carry

        lax.fori_loop(0, niter, body, 0)
        plsc.subcore_barrier()
        pltpu.sync_copy(shared.at[pl.ds(sid * rows_each, rows_each)], zv)
        pltpu.sync_copy(zv,
                        out_hbm.at[cid, pl.ds(sid * rows_each, rows_each)])

    return k(vals, batch, zeros2d)


def _seg_max_rows(vals, row, col, n):
    m = jax.ops.segment_max(vals[row], col, num_segments=n)
    return jnp.maximum(vals, m)


def kernel(x, edge_index, batch, params):
    n = x.shape[0]
    g = 64
    e = edge_index.shape[1]
    row, col = edge_index[0], edge_index[1]
    row2d = row.reshape(e // 128, 128)
    col2d = col.reshape(e // 128, 128)

    rows_each = ((n + _NS - 1) // _NS + 7) // 8 * 8
    zeros1d = jnp.zeros((rows_each,), jnp.float32)
    ones1d = jnp.ones((128,), jnp.float32)
    zeros2d = jnp.zeros((rows_each, x.shape[1] // 2), jnp.float32)
    zeros2p = jnp.zeros((64 // _NS, 256), jnp.float32)

    degp = _sc_deg(col2d, zeros1d, ones1d, n).reshape(_NC, n)
    dinv = _dinv_from_partials(degp[0].reshape(n, 1), degp[1].reshape(n, 1))

    h = x
    readout = None
    for i in range(1, 5):
        W, b = params[f'W{i}'], params[f'b{i}']
        lW, lb = params[f'lW{i}'], params[f'lb{i}']
        hws = _mm_scale(h, W, dinv)
        part = _sc_scatter_rows(hws.reshape(2 * n, hws.shape[1] // 2),
                                row2d, col2d, zeros2d, n)
        h_act = _post_gcn(part, hws, dinv, b)
        h = _seg_max_rows(h_act, row, col, n)
        if readout is None:
            readout = jnp.zeros((n, lW.shape[1]), jnp.float32)
        readout = _readout_add(h, lW, lb, readout)

    n_pad = -(-n // 128) * 128
    readout_pad = jnp.pad(readout, ((0, n_pad - n), (0, 0)))
    batch_pad = jnp.pad(batch, (0, n_pad - n))
    gp = _sc_batch_sum(readout_pad, batch_pad, zeros2p, g)
    return _classifier(gp[0], gp[1], params)


# speedup vs baseline: 4.7852x; 1.4636x over previous
import functools

import jax
import jax.numpy as jnp
from jax import lax
from jax.experimental import pallas as pl
from jax.experimental.pallas import tpu as pltpu
from jax.experimental.pallas import tpu_sc as plsc

_NC, _NS = 2, 16
_NW = _NC * _NS
_SC_MESH = dict(mesh=plsc.VectorSubcoreMesh(core_axis_name="c",
                                            subcore_axis_name="s"))

_SELU_A = 1.6732632423543772
_SELU_S = 1.0507009873554805
_BN_S = 1.0 / (1.00001 ** 0.5)


def _mm_scale_body(h_ref, w_ref, dinv_ref, o_ref):
    hw = jnp.dot(h_ref[...], w_ref[...], preferred_element_type=jnp.float32)
    o_ref[...] = hw * dinv_ref[...]


def _mm_scale(h, W, dinv2d, block=1000):
    n, _ = h.shape
    o = W.shape[1]
    return pl.pallas_call(
        _mm_scale_body,
        grid=(n // block,),
        in_specs=[
            pl.BlockSpec((block, h.shape[1]), lambda i: (i, 0)),
            pl.BlockSpec(W.shape, lambda i: (0, 0)),
            pl.BlockSpec((block, 1), lambda i: (i, 0)),
        ],
        out_specs=pl.BlockSpec((block, o), lambda i: (i, 0)),
        out_shape=jax.ShapeDtypeStruct((n, o), jnp.float32),
    )(h, W, dinv2d)


def _dinv_body(p0_ref, p1_ref, o_ref):
    o_ref[...] = lax.rsqrt(1.0 + p0_ref[...] + p1_ref[...])


def _dinv_from_partials(p0, p1, block=1000):
    n = p0.shape[0]
    return pl.pallas_call(
        _dinv_body,
        grid=(n // block,),
        in_specs=[pl.BlockSpec((block, 1), lambda i: (i, 0))] * 2,
        out_specs=pl.BlockSpec((block, 1), lambda i: (i, 0)),
        out_shape=jax.ShapeDtypeStruct((n, 1), jnp.float32),
    )(p0, p1)


def _post_gcn_body(part_ref, hws_ref, dinv_ref, b_ref, o_ref):
    part = jnp.concatenate([part_ref[0], part_ref[1]], axis=-1)
    z = dinv_ref[...] * (part + hws_ref[...]) + b_ref[...]
    neg = _SELU_A * (jnp.exp(jnp.minimum(z, 0.0)) - 1.0)
    o_ref[...] = _SELU_S * jnp.where(z > 0, z, neg)


def _post_gcn(part3d, hws, dinv2d, b, block=1000):
    n, hdim = hws.shape
    half = hdim // 2
    return pl.pallas_call(
        _post_gcn_body,
        grid=(n // block,),
        in_specs=[
            pl.BlockSpec((2, block, half), lambda i: (0, i, 0)),
            pl.BlockSpec((block, hdim), lambda i: (i, 0)),
            pl.BlockSpec((block, 1), lambda i: (i, 0)),
            pl.BlockSpec((1, hdim), lambda i: (0, 0)),
        ],
        out_specs=pl.BlockSpec((block, hdim), lambda i: (i, 0)),
        out_shape=jax.ShapeDtypeStruct((n, hdim), jnp.float32),
    )(part3d, hws, dinv2d, b.reshape(1, hdim))


def _readout_body(h_ref, w_ref, b_ref, r_ref, o_ref):
    z = jnp.dot(h_ref[...], w_ref[...], preferred_element_type=jnp.float32)
    z = z + b_ref[...]
    z = z - jnp.max(z, axis=-1, keepdims=True)
    e = jnp.exp(z)
    o_ref[...] = r_ref[...] + e / jnp.sum(e, axis=-1, keepdims=True)


def _readout_add(h, lW, lb, r, block=1000):
    n, hdim = h.shape
    p = lW.shape[1]
    return pl.pallas_call(
        _readout_body,
        grid=(n // block,),
        in_specs=[
            pl.BlockSpec((block, hdim), lambda i: (i, 0)),
            pl.BlockSpec(lW.shape, lambda i: (0, 0)),
            pl.BlockSpec((1, p), lambda i: (0, 0)),
            pl.BlockSpec((block, p), lambda i: (i, 0)),
        ],
        out_specs=pl.BlockSpec((block, p), lambda i: (i, 0)),
        out_shape=jax.ShapeDtypeStruct((n, p), jnp.float32),
    )(h, lW, lb.reshape(1, p), r)


def _cls_body(ga_ref, gb_ref, w1, b1, g1, be1, w2, b2, g2, be2, w3, b3, g3, be3,
              w4, b4, o_ref):
    def lin(v, w, b):
        return jnp.dot(v, w[...], preferred_element_type=jnp.float32) + b[...]

    def bn(v, ga, be):
        return v * _BN_S * ga[...] + be[...]

    z = bn(jnp.maximum(lin(ga_ref[...] + gb_ref[...], w1, b1), 0.0), g1, be1)
    z = bn(jnp.maximum(lin(z, w2, b2), 0.0), g2, be2)
    z = bn(jnp.maximum(lin(z, w3, b3), 0.0), g3, be3)
    logits = lin(z, w4, b4)
    p = 1.0 / (1.0 + jnp.exp(-logits))
    p = p - jnp.max(p, axis=-1, keepdims=True)
    e = jnp.exp(p)
    o_ref[...] = e / jnp.sum(e, axis=-1, keepdims=True)


def _classifier(ga, gb, params):
    P = params
    args = [ga, gb]
    for names in (('cW1', 'cb1', 'g1', 'be1'),
                  ('cW2', 'cb2', 'g2', 'be2'),
                  ('cW3', 'cb3', 'g3', 'be3')):
        w, b, gm, be = (P[k] for k in names)
        args += [w, b.reshape(1, -1), gm.reshape(1, -1), be.reshape(1, -1)]
    args += [P['cW4'], P['cb4'].reshape(1, -1)]
    nout = P['cW4'].shape[1]
    specs = [pl.BlockSpec(a.shape, lambda i, _s=a.shape: (0,) * len(_s))
             for a in args]
    return pl.pallas_call(
        _cls_body,
        grid=(1,),
        in_specs=specs,
        out_specs=pl.BlockSpec((ga.shape[0], nout), lambda i: (0, 0)),
        out_shape=jax.ShapeDtypeStruct((ga.shape[0], nout), jnp.float32),
    )(*args)


def _zero_shared_rows(zeros_v, shared, sid, nrows_each, nrows_last):
    @pl.when(sid < _NS - 1)
    def _():
        pltpu.sync_copy(zeros_v, shared.at[pl.ds(sid * nrows_each,
                                                 nrows_each)])

    @pl.when(sid == _NS - 1)
    def _():
        pltpu.sync_copy(zeros_v.at[pl.ds(0, nrows_last)],
                        shared.at[pl.ds((_NS - 1) * nrows_each, nrows_last)])


def _sc_deg(col2d, zeros1d, ones1d, n):
    nchunk = col2d.shape[0]
    niter = (nchunk + _NW - 1) // _NW
    rows_each = ((n + _NS - 1) // _NS + 7) // 8 * 8
    rows_last = n - (_NS - 1) * rows_each

    @functools.partial(
        pl.kernel,
        out_type=jax.ShapeDtypeStruct((_NC * n,), jnp.float32),
        scratch_types=[
            pltpu.VMEM((128,), jnp.int32),
            pltpu.VMEM((128,), jnp.float32),
            pltpu.VMEM((rows_each,), jnp.float32),
            pltpu.VMEM_SHARED((n,), jnp.float32),
        ],
        **_SC_MESH,
    )
    def k(col_hbm, zeros_hbm, ones_hbm, out_hbm, idx_v, ones_v, zv, shared):
        cid = lax.axis_index("c")
        sid = lax.axis_index("s")
        wid = sid * _NC + cid
        pltpu.sync_copy(ones_hbm, ones_v)
        pltpu.sync_copy(zeros_hbm, zv)
        _zero_shared_rows(zv, shared, sid, rows_each, rows_last)
        plsc.subcore_barrier()

        def body(r, carry):
            ci = wid + _NW * r

            @pl.when(ci < nchunk)
            def _():
                pltpu.sync_copy(col_hbm.at[ci], idx_v)
                pltpu.sync_copy(ones_v, shared.at[idx_v], add=True)
            return carry

        lax.fori_loop(0, niter, body, 0)
        plsc.subcore_barrier()

        @pl.when(sid < _NS - 1)
        def _():
            pltpu.sync_copy(shared.at[pl.ds(sid * rows_each, rows_each)], zv)
            pltpu.sync_copy(
                zv, out_hbm.at[pl.ds(cid * n + sid * rows_each, rows_each)])

        @pl.when(sid == _NS - 1)
        def _():
            sl = pl.ds(0, rows_last)
            pltpu.sync_copy(
                shared.at[pl.ds((_NS - 1) * rows_each, rows_last)], zv.at[sl])
            pltpu.sync_copy(
                zv.at[sl],
                out_hbm.at[pl.ds(cid * n + (_NS - 1) * rows_each, rows_last)])

    return k(col2d, zeros1d, ones1d)


def _sc_scatter_rows(vals2h, row2d, col2d, zeros2d, n):
    nchunk = row2d.shape[0]
    niter = (nchunk + _NS - 1) // _NS
    half = vals2h.shape[1]
    rows_each = ((n + _NS - 1) // _NS + 7) // 8 * 8
    rows_last = n - (_NS - 1) * rows_each

    @functools.partial(
        pl.kernel,
        out_type=jax.ShapeDtypeStruct((_NC, n, half), jnp.float32),
        scratch_types=[
            pltpu.VMEM((128,), jnp.int32),
            pltpu.VMEM((128,), jnp.int32),
            pltpu.VMEM((128, half), jnp.float32),
            pltpu.VMEM((rows_each, half), jnp.float32),
            pltpu.VMEM_SHARED((n, half), jnp.float32),
            pltpu.SemaphoreType.DMA,
        ],
        compiler_params=pltpu.CompilerParams(use_tc_tiling_on_sc=False),
        **_SC_MESH,
    )
    def k(vals_hbm, row_hbm, col_hbm, zeros_hbm, out_hbm,
          idxr, idxc, rows_v, zv, shared, sem):
        cid = lax.axis_index("c")
        sid = lax.axis_index("s")
        pltpu.sync_copy(zeros_hbm, zv)
        _zero_shared_rows(zv, shared, sid, rows_each, rows_last)
        plsc.subcore_barrier()

        def body(r, carry):
            ci = sid + _NS * r

            @pl.when(ci < nchunk)
            def _():
                pltpu.sync_copy(row_hbm.at[ci], idxr)
                for q in range(8):
                    sl = pl.ds(q * 16, 16)
                    idxr[sl] = idxr[sl] * 2 + cid
                pltpu.async_copy(vals_hbm.at[idxr], rows_v, sem).wait()
                pltpu.sync_copy(col_hbm.at[ci], idxc)
                pltpu.sync_copy(rows_v, shared.at[idxc], add=True)
            return carry

        lax.fori_loop(0, niter, body, 0)
        plsc.subcore_barrier()

        @pl.when(sid < _NS - 1)
        def _():
            pltpu.sync_copy(shared.at[pl.ds(sid * rows_each, rows_each)], zv)
            pltpu.sync_copy(
                zv, out_hbm.at[cid, pl.ds(sid * rows_each, rows_each)])

        @pl.when(sid == _NS - 1)
        def _():
            sl = pl.ds(0, rows_last)
            pltpu.sync_copy(
                shared.at[pl.ds((_NS - 1) * rows_each, rows_last)], zv.at[sl])
            pltpu.sync_copy(
                zv.at[sl],
                out_hbm.at[cid, pl.ds((_NS - 1) * rows_each, rows_last)])

    return k(vals2h, row2d, col2d, zeros2d)


def _sc_batch_sum(vals, batch, zeros2d, g):
    n, p = vals.shape
    chunk = 128
    nchunk = n // chunk
    niter = (nchunk + _NW - 1) // _NW
    rows_each = g // _NS

    @functools.partial(
        pl.kernel,
        out_type=jax.ShapeDtypeStruct((_NC, g, p), jnp.float32),
        scratch_types=[
            pltpu.VMEM((chunk,), jnp.int32),
            pltpu.VMEM((chunk, p), jnp.float32),
            pltpu.VMEM((rows_each, p), jnp.float32),
            pltpu.VMEM_SHARED((g, p), jnp.float32),
        ],
        compiler_params=pltpu.CompilerParams(use_tc_tiling_on_sc=False),
        **_SC_MESH,
    )
    def k(vals_hbm, batch_hbm, zeros_hbm, out_hbm, idxb, val_v, zv, shared):
        cid = lax.axis_index("c")
        sid = lax.axis_index("s")
        wid = sid * _NC + cid
        pltpu.sync_copy(zeros_hbm, zv)
        pltpu.sync_copy(zv, shared.at[pl.ds(sid * rows_each, rows_each)])
        plsc.subcore_barrier()

        def body(r, carry):
            ci = wid + _NW * r

            @pl.when(ci < nchunk)
            def _():
                pltpu.sync_copy(batch_hbm.at[pl.ds(ci * chunk, chunk)], idxb)
                pltpu.sync_copy(vals_hbm.at[pl.ds(ci * chunk, chunk)], val_v)
                pltpu.sync_copy(val_v, shared.at[idxb], add=True)
            return carry

        lax.fori_loop(0, niter, body, 0)
        plsc.subcore_barrier()
        pltpu.sync_copy(shared.at[pl.ds(sid * rows_each, rows_each)], zv)
        pltpu.sync_copy(zv,
                        out_hbm.at[cid, pl.ds(sid * rows_each, rows_each)])

    return k(vals, batch, zeros2d)


_OWN = 313
_OWN_MUL, _OWN_SHIFT = 13401, 22
_REG = 10240
_PACK_SHIFT = 14


def _iota16():
    return lax.iota(jnp.int32, 16)


def _take16(x, idx):
    dnums = lax.GatherDimensionNumbers(offset_dims=(), collapsed_slice_dims=(0,),
                                       start_index_map=(0,))
    return lax.gather(x, idx[:, None], dnums, (1,),
                      mode=lax.GatherScatterMode.PROMISE_IN_BOUNDS)


def _sc_partition(row2d, col2d):
    nchunk = row2d.shape[0]
    niter = (nchunk + _NW - 1) // _NW

    @functools.partial(
        pl.kernel,
        out_type=(jax.ShapeDtypeStruct((_NW * _REG,), jnp.int32),
                  jax.ShapeDtypeStruct((_NW * _NW,), jnp.int32)),
        scratch_types=[
            pltpu.VMEM((128,), jnp.int32),
            pltpu.VMEM((128,), jnp.int32),
            pltpu.VMEM((_REG,), jnp.int32),
            pltpu.VMEM((_NW,), jnp.int32),
            pltpu.VMEM((_NW,), jnp.int32),
        ],
        compiler_params=pltpu.CompilerParams(use_tc_tiling_on_sc=False,
                                             needs_layout_passes=False),
        **_SC_MESH,
    )
    def k(row_hbm, col_hbm, parts_hbm, counts_hbm,
          cbuf, rbuf, outbuf, bins, wptr):
        cid = lax.axis_index("c")
        sid = lax.axis_index("s")
        w = sid * _NC + cid
        it16 = _iota16()
        zero16 = jnp.zeros((16,), jnp.int32)
        bins[pl.ds(0, 16)] = zero16
        bins[pl.ds(16, 16)] = zero16

        def owner_of(col16):
            return lax.shift_right_logical(col16 * _OWN_MUL, _OWN_SHIFT)

        def runs(os):
            prev = _take16(os, jnp.maximum(it16 - 1, 0))
            nxt = _take16(os, jnp.minimum(it16 + 1, 15))
            is_start = (it16 == 0) | (os != prev)
            is_end = (it16 == 15) | (os != nxt)
            run_base = plsc.cummax(jnp.where(is_start, it16, 0))
            rank = it16 - run_base
            return rank, is_end

        def count_body(r, carry):
            ci = w + _NW * r

            @pl.when(ci < nchunk)
            def _():
                pltpu.sync_copy(col_hbm.at[ci], cbuf)
                for q in range(8):
                    col16 = cbuf[pl.ds(q * 16, 16)]
                    os, _unused = plsc.sort_key_val(owner_of(col16), it16)
                    rank, is_end = runs(os)
                    plsc.addupdate_scatter(bins, [os], rank + 1, mask=is_end)
            return carry

        lax.fori_loop(0, niter, count_body, 0)
        pltpu.sync_copy(bins, counts_hbm.at[pl.ds(w * _NW, _NW)])

        b0 = bins[pl.ds(0, 16)]
        b1 = bins[pl.ds(16, 16)]
        c0 = plsc.cumsum(b0)
        t0 = jnp.max(c0)
        wptr[pl.ds(0, 16)] = c0 - b0
        wptr[pl.ds(16, 16)] = plsc.cumsum(b1) - b1 + t0

        def place_body(r, carry):
            ci = w + _NW * r

            @pl.when(ci < nchunk)
            def _():
                pltpu.sync_copy(col_hbm.at[ci], cbuf)
                pltpu.sync_copy(row_hbm.at[ci], rbuf)
                for q in range(8):
                    col16 = cbuf[pl.ds(q * 16, 16)]
                    row16 = rbuf[pl.ds(q * 16, 16)]
                    os, perm = plsc.sort_key_val(owner_of(col16), it16)
                    row_s = _take16(row16, perm)
                    col_s = _take16(col16, perm)
                    rank, is_end = runs(os)
                    base = plsc.load_gather(wptr, [os])
                    packed = row_s | lax.shift_left(col_s - os * _OWN,
                                                    _PACK_SHIFT)
                    plsc.store_scatter(outbuf, [base + rank], packed)
                    plsc.addupdate_scatter(wptr, [os], rank + 1, mask=is_end)
            return carry

        lax.fori_loop(0, niter, place_body, 0)

        nfull = nchunk // _NW

        @pl.when(w < nchunk - nfull * _NW)
        def _():
            for q in range((_REG - (nfull + 1) * 128) // 16):
                outbuf[pl.ds((nfull + 1) * 128 + q * 16, 16)] = zero16

        @pl.when(w >= nchunk - nfull * _NW)
        def _():
            for q in range((_REG - nfull * 128) // 16):
                outbuf[pl.ds(nfull * 128 + q * 16, 16)] = zero16

        pltpu.sync_copy(outbuf, parts_hbm.at[pl.ds(w * _REG, _REG)])

    return k(row2d, col2d)


def _sc_seg_max(hact, parts, counts, n):
    hdim = hact.shape[1]
    npw_last = n - (_NW - 1) * _OWN

    @functools.partial(
        pl.kernel,
        out_type=jax.ShapeDtypeStruct((n, hdim), jnp.float32),
        scratch_types=[
            pltpu.VMEM((320, hdim), jnp.float32),
            pltpu.VMEM((_NW * _NW,), jnp.int32),
            pltpu.VMEM((128,), jnp.int32),
            pltpu.VMEM((128,), jnp.int32),
            pltpu.VMEM((128, hdim), jnp.float32),
            pltpu.SemaphoreType.DMA,
        ],
        compiler_params=pltpu.CompilerParams(use_tc_tiling_on_sc=False,
                                             needs_layout_passes=False),
        **_SC_MESH,
    )
    def k(hact_hbm, parts_hbm, counts_hbm, out_hbm,
          acc, cntm, pbuf, ridx, grow, sem):
        cid = lax.axis_index("c")
        sid = lax.axis_index("s")
        w = sid * _NC + cid
        it16 = _iota16()
        pltpu.sync_copy(counts_hbm, cntm)

        @pl.when(w < _NW - 1)
        def _():
            pltpu.sync_copy(hact_hbm.at[pl.ds(w * _OWN, _OWN)],
                            acc.at[pl.ds(0, _OWN)])

        @pl.when(w == _NW - 1)
        def _():
            pltpu.sync_copy(hact_hbm.at[pl.ds((_NW - 1) * _OWN, npw_last)],
                            acc.at[pl.ds(0, npw_last)])

        def src_body(v, carry):
            vbase = pl.multiple_of(v * _NW, _NW)
            a = cntm[pl.ds(vbase, 16)]
            b = cntm[pl.ds(vbase + 16, 16)]
            off = (jnp.sum(jnp.where(it16 < w, a, 0))
                   + jnp.sum(jnp.where(it16 < w - 16, b, 0)))
            cnt = (jnp.sum(jnp.where(it16 == w, a, 0))
                   + jnp.sum(jnp.where(it16 == w - 16, b, 0)))
            base = v * _REG + off
            st = lax.shift_left(lax.shift_right_logical(base, 3), 3)
            nblk = (base + cnt - st + 127) // 128
            head = base - st

            def blk_body(j, carry2):
                boff = pl.multiple_of(st + j * 128, 8)
                pltpu.sync_copy(parts_hbm.at[pl.ds(boff, 128)], pbuf)
                for q in range(8):
                    pv = pbuf[pl.ds(q * 16, 16)]
                    ridx[pl.ds(q * 16, 16)] = pv & ((1 << _PACK_SHIFT) - 1)
                pltpu.async_copy(hact_hbm.at[ridx], grow, sem).wait()

                def q_body(q, carry3):
                    pv = plsc.load_gather(pbuf, [q * 16 + it16])
                    cloc = lax.shift_right_logical(pv, _PACK_SHIFT)
                    rel = j * 128 + q * 16 + it16 - head
                    valid = ((rel >= 0) & (rel < cnt)).astype(jnp.int32)
                    for l in range(16):
                        lsel = jnp.full((16,), l, jnp.int32)
                        cb = _take16(cloc, lsel)
                        mb = _take16(valid, lsel) != 0
                        rb = jnp.full((16,), q * 16 + l, jnp.int32)
                        for j2 in range(hdim // 16):
                            cols = it16 + j2 * 16
                            av = plsc.load_gather(acc, [cb, cols])
                            gv = plsc.load_gather(grow, [rb, cols])
                            plsc.store_scatter(acc, [cb, cols],
                                               jnp.maximum(av, gv), mask=mb)
                    return carry3

                lax.fori_loop(0, 8, q_body, 0)
                return carry2

            lax.fori_loop(0, nblk, blk_body, 0)
            return carry

        lax.fori_loop(0, _NW, src_body, 0)

        @pl.when(w < _NW - 1)
        def _():
            pltpu.sync_copy(acc.at[pl.ds(0, _OWN)],
                            out_hbm.at[pl.ds(w * _OWN, _OWN)])

        @pl.when(w == _NW - 1)
        def _():
            pltpu.sync_copy(acc.at[pl.ds(0, npw_last)],
                            out_hbm.at[pl.ds((_NW - 1) * _OWN, npw_last)])

    return k(hact, parts, counts)


def kernel(x, edge_index, batch, params):
    n = x.shape[0]
    g = 64
    e = edge_index.shape[1]
    row, col = edge_index[0], edge_index[1]
    row2d = row.reshape(e // 128, 128)
    col2d = col.reshape(e // 128, 128)

    rows_each = ((n + _NS - 1) // _NS + 7) // 8 * 8
    zeros1d = jnp.zeros((rows_each,), jnp.float32)
    ones1d = jnp.ones((128,), jnp.float32)
    zeros2d = jnp.zeros((rows_each, x.shape[1] // 2), jnp.float32)
    zeros2p = jnp.zeros((64 // _NS, 256), jnp.float32)

    degp = _sc_deg(col2d, zeros1d, ones1d, n).reshape(_NC, n)
    dinv = _dinv_from_partials(degp[0].reshape(n, 1), degp[1].reshape(n, 1))
    parts, counts = _sc_partition(row2d, col2d)

    h = x
    readout = None
    for i in range(1, 5):
        W, b = params[f'W{i}'], params[f'b{i}']
        lW, lb = params[f'lW{i}'], params[f'lb{i}']
        hws = _mm_scale(h, W, dinv)
        part = _sc_scatter_rows(hws.reshape(2 * n, hws.shape[1] // 2),
                                row2d, col2d, zeros2d, n)
        h_act = _post_gcn(part, hws, dinv, b)
        h = _sc_seg_max(h_act, parts, counts, n)
        if readout is None:
            readout = jnp.zeros((n, lW.shape[1]), jnp.float32)
        readout = _readout_add(h, lW, lb, readout)

    n_pad = -(-n // 128) * 128
    readout_pad = jnp.pad(readout, ((0, n_pad - n), (0, 0)))
    batch_pad = jnp.pad(batch, (0, n_pad - n))
    gp = _sc_batch_sum(readout_pad, batch_pad, zeros2p, g)
    return _classifier(gp[0], gp[1], params)


# speedup vs baseline: 5.6799x; 1.1870x over previous
import functools

import jax
import jax.numpy as jnp
from jax import lax
from jax.experimental import pallas as pl
from jax.experimental.pallas import tpu as pltpu
from jax.experimental.pallas import tpu_sc as plsc

_NC, _NS = 2, 16
_NW = _NC * _NS
_SC_MESH = dict(mesh=plsc.VectorSubcoreMesh(core_axis_name="c",
                                            subcore_axis_name="s"))

_SELU_A = 1.6732632423543772
_SELU_S = 1.0507009873554805
_BN_S = 1.0 / (1.00001 ** 0.5)


def _mm_scale_body(h_ref, w_ref, dinv_ref, o_ref):
    hw = jnp.dot(h_ref[...], w_ref[...], preferred_element_type=jnp.float32)
    o_ref[...] = hw * dinv_ref[...]


def _mm_scale(h, W, dinv2d, block=1000):
    n, _ = h.shape
    o = W.shape[1]
    return pl.pallas_call(
        _mm_scale_body,
        grid=(n // block,),
        in_specs=[
            pl.BlockSpec((block, h.shape[1]), lambda i: (i, 0)),
            pl.BlockSpec(W.shape, lambda i: (0, 0)),
            pl.BlockSpec((block, 1), lambda i: (i, 0)),
        ],
        out_specs=pl.BlockSpec((block, o), lambda i: (i, 0)),
        out_shape=jax.ShapeDtypeStruct((n, o), jnp.float32),
    )(h, W, dinv2d)


def _dinv_body(p0_ref, p1_ref, o_ref):
    o_ref[...] = lax.rsqrt(1.0 + p0_ref[...] + p1_ref[...])


def _dinv_from_partials(p0, p1, block=1000):
    n = p0.shape[0]
    return pl.pallas_call(
        _dinv_body,
        grid=(n // block,),
        in_specs=[pl.BlockSpec((block, 1), lambda i: (i, 0))] * 2,
        out_specs=pl.BlockSpec((block, 1), lambda i: (i, 0)),
        out_shape=jax.ShapeDtypeStruct((n, 1), jnp.float32),
    )(p0, p1)


def _post_gcn_body(part_ref, hws_ref, dinv_ref, b_ref, o_ref):
    part = jnp.concatenate([part_ref[0], part_ref[1]], axis=-1)
    z = dinv_ref[...] * (part + hws_ref[...]) + b_ref[...]
    neg = _SELU_A * (jnp.exp(jnp.minimum(z, 0.0)) - 1.0)
    o_ref[...] = _SELU_S * jnp.where(z > 0, z, neg)


def _post_gcn(part3d, hws, dinv2d, b, block=1000):
    n, hdim = hws.shape
    half = hdim // 2
    return pl.pallas_call(
        _post_gcn_body,
        grid=(n // block,),
        in_specs=[
            pl.BlockSpec((2, block, half), lambda i: (0, i, 0)),
            pl.BlockSpec((block, hdim), lambda i: (i, 0)),
            pl.BlockSpec((block, 1), lambda i: (i, 0)),
            pl.BlockSpec((1, hdim), lambda i: (0, 0)),
        ],
        out_specs=pl.BlockSpec((block, hdim), lambda i: (i, 0)),
        out_shape=jax.ShapeDtypeStruct((n, hdim), jnp.float32),
    )(part3d, hws, dinv2d, b.reshape(1, hdim))


def _readout_body(h_ref, w_ref, b_ref, r_ref, o_ref):
    z = jnp.dot(h_ref[...], w_ref[...], preferred_element_type=jnp.float32)
    z = z + b_ref[...]
    z = z - jnp.max(z, axis=-1, keepdims=True)
    e = jnp.exp(z)
    o_ref[...] = r_ref[...] + e / jnp.sum(e, axis=-1, keepdims=True)


def _readout_add(h, lW, lb, r, block=1000):
    n, hdim = h.shape
    p = lW.shape[1]
    return pl.pallas_call(
        _readout_body,
        grid=(n // block,),
        in_specs=[
            pl.BlockSpec((block, hdim), lambda i: (i, 0)),
            pl.BlockSpec(lW.shape, lambda i: (0, 0)),
            pl.BlockSpec((1, p), lambda i: (0, 0)),
            pl.BlockSpec((block, p), lambda i: (i, 0)),
        ],
        out_specs=pl.BlockSpec((block, p), lambda i: (i, 0)),
        out_shape=jax.ShapeDtypeStruct((n, p), jnp.float32),
    )(h, lW, lb.reshape(1, p), r)


def _cls_body(ga_ref, gb_ref, w1, b1, g1, be1, w2, b2, g2, be2, w3, b3, g3, be3,
              w4, b4, o_ref):
    def lin(v, w, b):
        return jnp.dot(v, w[...], preferred_element_type=jnp.float32) + b[...]

    def bn(v, ga, be):
        return v * _BN_S * ga[...] + be[...]

    z = bn(jnp.maximum(lin(ga_ref[...] + gb_ref[...], w1, b1), 0.0), g1, be1)
    z = bn(jnp.maximum(lin(z, w2, b2), 0.0), g2, be2)
    z = bn(jnp.maximum(lin(z, w3, b3), 0.0), g3, be3)
    logits = lin(z, w4, b4)
    p = 1.0 / (1.0 + jnp.exp(-logits))
    p = p - jnp.max(p, axis=-1, keepdims=True)
    e = jnp.exp(p)
    o_ref[...] = e / jnp.sum(e, axis=-1, keepdims=True)


def _classifier(ga, gb, params):
    P = params
    args = [ga, gb]
    for names in (('cW1', 'cb1', 'g1', 'be1'),
                  ('cW2', 'cb2', 'g2', 'be2'),
                  ('cW3', 'cb3', 'g3', 'be3')):
        w, b, gm, be = (P[k] for k in names)
        args += [w, b.reshape(1, -1), gm.reshape(1, -1), be.reshape(1, -1)]
    args += [P['cW4'], P['cb4'].reshape(1, -1)]
    nout = P['cW4'].shape[1]
    specs = [pl.BlockSpec(a.shape, lambda i, _s=a.shape: (0,) * len(_s))
             for a in args]
    return pl.pallas_call(
        _cls_body,
        grid=(1,),
        in_specs=specs,
        out_specs=pl.BlockSpec((ga.shape[0], nout), lambda i: (0, 0)),
        out_shape=jax.ShapeDtypeStruct((ga.shape[0], nout), jnp.float32),
    )(*args)


def _zero_shared_rows(zeros_v, shared, sid, nrows_each, nrows_last):
    @pl.when(sid < _NS - 1)
    def _():
        pltpu.sync_copy(zeros_v, shared.at[pl.ds(sid * nrows_each,
                                                 nrows_each)])

    @pl.when(sid == _NS - 1)
    def _():
        pltpu.sync_copy(zeros_v.at[pl.ds(0, nrows_last)],
                        shared.at[pl.ds((_NS - 1) * nrows_each, nrows_last)])


def _sc_deg(col2d, zeros1d, ones1d, n):
    nchunk = col2d.shape[0]
    niter = (nchunk + _NW - 1) // _NW
    rows_each = ((n + _NS - 1) // _NS + 7) // 8 * 8
    rows_last = n - (_NS - 1) * rows_each

    @functools.partial(
        pl.kernel,
        out_type=jax.ShapeDtypeStruct((_NC * n,), jnp.float32),
        scratch_types=[
            pltpu.VMEM((128,), jnp.int32),
            pltpu.VMEM((128,), jnp.float32),
            pltpu.VMEM((rows_each,), jnp.float32),
            pltpu.VMEM_SHARED((n,), jnp.float32),
        ],
        **_SC_MESH,
    )
    def k(col_hbm, zeros_hbm, ones_hbm, out_hbm, idx_v, ones_v, zv, shared):
        cid = lax.axis_index("c")
        sid = lax.axis_index("s")
        wid = sid * _NC + cid
        pltpu.sync_copy(ones_hbm, ones_v)
        pltpu.sync_copy(zeros_hbm, zv)
        _zero_shared_rows(zv, shared, sid, rows_each, rows_last)
        plsc.subcore_barrier()

        def body(r, carry):
            ci = wid + _NW * r

            @pl.when(ci < nchunk)
            def _():
                pltpu.sync_copy(col_hbm.at[ci], idx_v)
                pltpu.sync_copy(ones_v, shared.at[idx_v], add=True)
            return carry

        lax.fori_loop(0, niter, body, 0)
        plsc.subcore_barrier()

        @pl.when(sid < _NS - 1)
        def _():
            pltpu.sync_copy(shared.at[pl.ds(sid * rows_each, rows_each)], zv)
            pltpu.sync_copy(
                zv, out_hbm.at[pl.ds(cid * n + sid * rows_each, rows_each)])

        @pl.when(sid == _NS - 1)
        def _():
            sl = pl.ds(0, rows_last)
            pltpu.sync_copy(
                shared.at[pl.ds((_NS - 1) * rows_each, rows_last)], zv.at[sl])
            pltpu.sync_copy(
                zv.at[sl],
                out_hbm.at[pl.ds(cid * n + (_NS - 1) * rows_each, rows_last)])

    return k(col2d, zeros1d, ones1d)


_NBUF = 4


def _sc_scatter_rows(vals2p, row2p, col2p, zeros2d, n):
    nchunk = row2p.shape[0]
    assert nchunk % _NS == 0
    niter = nchunk // _NS
    half = vals2p.shape[1]
    rows_each = ((n + _NS - 1) // _NS + 7) // 8 * 8
    rows_last = n - (_NS - 1) * rows_each

    @functools.partial(
        pl.kernel,
        out_type=jax.ShapeDtypeStruct((_NC, n, half), jnp.float32),
        scratch_types=[
            pltpu.VMEM((_NBUF, 128), jnp.int32),
            pltpu.VMEM((_NBUF, 128), jnp.int32),
            pltpu.VMEM((_NBUF, 128, half), jnp.float32),
            pltpu.VMEM((rows_each, half), jnp.float32),
            pltpu.VMEM_SHARED((n, half), jnp.float32),
            pltpu.SemaphoreType.DMA((_NBUF,)),
            pltpu.SemaphoreType.DMA((_NBUF,)),
            pltpu.SemaphoreType.DMA((_NBUF,)),
            pltpu.SemaphoreType.DMA((_NBUF,)),
        ],
        compiler_params=pltpu.CompilerParams(use_tc_tiling_on_sc=False),
        **_SC_MESH,
    )
    def k(vals_hbm, row_hbm, col_hbm, zeros_hbm, out_hbm,
          idxr, idxc, rows_v, zv, shared, semr, semc, semg, sems):
        cid = lax.axis_index("c")
        sid = lax.axis_index("s")
        pltpu.sync_copy(zeros_hbm, zv)
        _zero_shared_rows(zv, shared, sid, rows_each, rows_last)
        plsc.subcore_barrier()
        myvals = vals_hbm.at[pl.ds(cid, 2 * n + 1)]

        hidx, hg, hs = {}, {}, {}
        for t in range(niter + 2):
            if t < niter:
                b = t % _NBUF
                if t >= _NBUF:
                    hs[t - _NBUF].wait()
                ci = sid + _NS * t
                hidx[t] = (
                    pltpu.async_copy(row_hbm.at[ci], idxr.at[b], semr.at[b]),
                    pltpu.async_copy(col_hbm.at[ci], idxc.at[b], semc.at[b]))
            if 1 <= t < niter + 1:
                u, b = t - 1, (t - 1) % _NBUF
                hidx[u][0].wait()
                hg[u] = pltpu.async_copy(myvals.at[idxr.at[b]], rows_v.at[b],
                                         semg.at[b])
            if t >= 2:
                u, b = t - 2, (t - 2) % _NBUF
                hg[u].wait()
                hidx[u][1].wait()
                hs[u] = pltpu.async_copy(rows_v.at[b], shared.at[idxc.at[b]],
                                         sems.at[b], add=True)
        for t in range(max(0, niter - _NBUF), niter):
            hs[t].wait()
        plsc.subcore_barrier()

        @pl.when(sid < _NS - 1)
        def _():
            pltpu.sync_copy(shared.at[pl.ds(sid * rows_each, rows_each)], zv)
            pltpu.sync_copy(
                zv, out_hbm.at[cid, pl.ds(sid * rows_each, rows_each)])

        @pl.when(sid == _NS - 1)
        def _():
            sl = pl.ds(0, rows_last)
            pltpu.sync_copy(
                shared.at[pl.ds((_NS - 1) * rows_each, rows_last)], zv.at[sl])
            pltpu.sync_copy(
                zv.at[sl],
                out_hbm.at[cid, pl.ds((_NS - 1) * rows_each, rows_last)])

    return k(vals2p, row2p, col2p, zeros2d)


def _sc_batch_sum(vals, batch, zeros2d, g):
    n, p = vals.shape
    chunk = 128
    nchunk = n // chunk
    niter = (nchunk + _NW - 1) // _NW
    rows_each = g // _NS

    @functools.partial(
        pl.kernel,
        out_type=jax.ShapeDtypeStruct((_NC, g, p), jnp.float32),
        scratch_types=[
            pltpu.VMEM((chunk,), jnp.int32),
            pltpu.VMEM((chunk, p), jnp.float32),
            pltpu.VMEM((rows_each, p), jnp.float32),
            pltpu.VMEM_SHARED((g, p), jnp.float32),
        ],
        compiler_params=pltpu.CompilerParams(use_tc_tiling_on_sc=False),
        **_SC_MESH,
    )
    def k(vals_hbm, batch_hbm, zeros_hbm, out_hbm, idxb, val_v, zv, shared):
        cid = lax.axis_index("c")
        sid = lax.axis_index("s")
        wid = sid * _NC + cid
        pltpu.sync_copy(zeros_hbm, zv)
        pltpu.sync_copy(zv, shared.at[pl.ds(sid * rows_each, rows_each)])
        plsc.subcore_barrier()

        def body(r, carry):
            ci = wid + _NW * r

            @pl.when(ci < nchunk)
            def _():
                pltpu.sync_copy(batch_hbm.at[pl.ds(ci * chunk, chunk)], idxb)
                pltpu.sync_copy(vals_hbm.at[pl.ds(ci * chunk, chunk)], val_v)
                pltpu.sync_copy(val_v, shared.at[idxb], add=True)
            return carry

        lax.fori_loop(0, niter, body, 0)
        plsc.subcore_barrier()
        pltpu.sync_copy(shared.at[pl.ds(sid * rows_each, rows_each)], zv)
        pltpu.sync_copy(zv,
                        out_hbm.at[cid, pl.ds(sid * rows_each, rows_each)])

    return k(vals, batch, zeros2d)


_OWN = 313
_OWN_MUL, _OWN_SHIFT = 13401, 22
_REG = 10240
_PACK_SHIFT = 14


def _iota16():
    return lax.iota(jnp.int32, 16)


def _take16(x, idx):
    dnums = lax.GatherDimensionNumbers(offset_dims=(), collapsed_slice_dims=(0,),
                                       start_index_map=(0,))
    return lax.gather(x, idx[:, None], dnums, (1,),
                      mode=lax.GatherScatterMode.PROMISE_IN_BOUNDS)


def _sc_partition(row2d, col2d):
    nchunk = row2d.shape[0]
    niter = (nchunk + _NW - 1) // _NW

    @functools.partial(
        pl.kernel,
        out_type=(jax.ShapeDtypeStruct((_NW * _REG,), jnp.int32),
                  jax.ShapeDtypeStruct((_NW * _NW,), jnp.int32)),
        scratch_types=[
            pltpu.VMEM((128,), jnp.int32),
            pltpu.VMEM((128,), jnp.int32),
            pltpu.VMEM((_REG,), jnp.int32),
            pltpu.VMEM((_NW,), jnp.int32),
            pltpu.VMEM((_NW,), jnp.int32),
        ],
        compiler_params=pltpu.CompilerParams(use_tc_tiling_on_sc=False,
                                             needs_layout_passes=False),
        **_SC_MESH,
    )
    def k(row_hbm, col_hbm, parts_hbm, counts_hbm,
          cbuf, rbuf, outbuf, bins, wptr):
        cid = lax.axis_index("c")
        sid = lax.axis_index("s")
        w = sid * _NC + cid
        it16 = _iota16()
        zero16 = jnp.zeros((16,), jnp.int32)
        bins[pl.ds(0, 16)] = zero16
        bins[pl.ds(16, 16)] = zero16

        def owner_of(col16):
            return lax.shift_right_logical(col16 * _OWN_MUL, _OWN_SHIFT)

        def runs(os):
            prev = _take16(os, jnp.maximum(it16 - 1, 0))
            nxt = _take16(os, jnp.minimum(it16 + 1, 15))
            is_start = (it16 == 0) | (os != prev)
            is_end = (it16 == 15) | (os != nxt)
            run_base = plsc.cummax(jnp.where(is_start, it16, 0))
            rank = it16 - run_base
            return rank, is_end

        def count_body(r, carry):
            ci = w + _NW * r

            @pl.when(ci < nchunk)
            def _():
                pltpu.sync_copy(col_hbm.at[ci], cbuf)
                for q in range(8):
                    col16 = cbuf[pl.ds(q * 16, 16)]
                    os, _unused = plsc.sort_key_val(owner_of(col16), it16)
                    rank, is_end = runs(os)
                    plsc.addupdate_scatter(bins, [os], rank + 1, mask=is_end)
            return carry

        lax.fori_loop(0, niter, count_body, 0)
        pltpu.sync_copy(bins, counts_hbm.at[pl.ds(w * _NW, _NW)])

        b0 = bins[pl.ds(0, 16)]
        b1 = bins[pl.ds(16, 16)]
        c0 = plsc.cumsum(b0)
        t0 = jnp.max(c0)
        wptr[pl.ds(0, 16)] = c0 - b0
        wptr[pl.ds(16, 16)] = plsc.cumsum(b1) - b1 + t0

        def place_body(r, carry):
            ci = w + _NW * r

            @pl.when(ci < nchunk)
            def _():
                pltpu.sync_copy(col_hbm.at[ci], cbuf)
                pltpu.sync_copy(row_hbm.at[ci], rbuf)
                for q in range(8):
                    col16 = cbuf[pl.ds(q * 16, 16)]
                    row16 = rbuf[pl.ds(q * 16, 16)]
                    os, perm = plsc.sort_key_val(owner_of(col16), it16)
                    row_s = _take16(row16, perm)
                    col_s = _take16(col16, perm)
                    rank, is_end = runs(os)
                    base = plsc.load_gather(wptr, [os])
                    packed = row_s | lax.shift_left(col_s - os * _OWN,
                                                    _PACK_SHIFT)
                    plsc.store_scatter(outbuf, [base + rank], packed)
                    plsc.addupdate_scatter(wptr, [os], rank + 1, mask=is_end)
            return carry

        lax.fori_loop(0, niter, place_body, 0)

        nfull = nchunk // _NW

        @pl.when(w < nchunk - nfull * _NW)
        def _():
            for q in range((_REG - (nfull + 1) * 128) // 16):
                outbuf[pl.ds((nfull + 1) * 128 + q * 16, 16)] = zero16

        @pl.when(w >= nchunk - nfull * _NW)
        def _():
            for q in range((_REG - nfull * 128) // 16):
                outbuf[pl.ds(nfull * 128 + q * 16, 16)] = zero16

        pltpu.sync_copy(outbuf, parts_hbm.at[pl.ds(w * _REG, _REG)])

    return k(row2d, col2d)


def _sc_seg_max(hact, parts, counts, n):
    hdim = hact.shape[1]
    npw_last = n - (_NW - 1) * _OWN

    @functools.partial(
        pl.kernel,
        out_type=jax.ShapeDtypeStruct((n, hdim), jnp.float32),
        scratch_types=[
            pltpu.VMEM((320, hdim), jnp.float32),
            pltpu.VMEM((_NW * _NW,), jnp.int32),
            pltpu.VMEM((128,), jnp.int32),
            pltpu.VMEM((128,), jnp.int32),
            pltpu.VMEM((128, hdim), jnp.float32),
            pltpu.SemaphoreType.DMA,
        ],
        compiler_params=pltpu.CompilerParams(use_tc_tiling_on_sc=False,
                                             needs_layout_passes=False),
        **_SC_MESH,
    )
    def k(hact_hbm, parts_hbm, counts_hbm, out_hbm,
          acc, cntm, pbuf, ridx, grow, sem):
        cid = lax.axis_index("c")
        sid = lax.axis_index("s")
        w = sid * _NC + cid
        it16 = _iota16()
        pltpu.sync_copy(counts_hbm, cntm)

        @pl.when(w < _NW - 1)
        def _():
            pltpu.sync_copy(hact_hbm.at[pl.ds(w * _OWN, _OWN)],
                            acc.at[pl.ds(0, _OWN)])

        @pl.when(w == _NW - 1)
        def _():
            pltpu.sync_copy(hact_hbm.at[pl.ds((_NW - 1) * _OWN, npw_last)],
                            acc.at[pl.ds(0, npw_last)])

        def src_body(v, carry):
            vbase = pl.multiple_of(v * _NW, _NW)
            a = cntm[pl.ds(vbase, 16)]
            b = cntm[pl.ds(vbase + 16, 16)]
            off = (jnp.sum(jnp.where(it16 < w, a, 0))
                   + jnp.sum(jnp.where(it16 < w - 16, b, 0)))
            cnt = (jnp.sum(jnp.where(it16 == w, a, 0))
                   + jnp.sum(jnp.where(it16 == w - 16, b, 0)))
            base = v * _REG + off
            st = lax.shift_left(lax.shift_right_logical(base, 3), 3)
            nblk = (base + cnt - st + 127) // 128
            head = base - st

            def blk_body(j, carry2):
                boff = pl.multiple_of(st + j * 128, 8)
                pltpu.sync_copy(parts_hbm.at[pl.ds(boff, 128)], pbuf)
                for q in range(8):
                    pv = pbuf[pl.ds(q * 16, 16)]
                    ridx[pl.ds(q * 16, 16)] = pv & ((1 << _PACK_SHIFT) - 1)
                pltpu.async_copy(hact_hbm.at[ridx], grow, sem).wait()

                def q_body(q, carry3):
                    pv = plsc.load_gather(pbuf, [q * 16 + it16])
                    cloc = lax.shift_right_logical(pv, _PACK_SHIFT)
                    rel = j * 128 + q * 16 + it16 - head
                    valid = ((rel >= 0) & (rel < cnt)).astype(jnp.int32)
                    for l in range(16):
                        lsel = jnp.full((16,), l, jnp.int32)
                        cb = _take16(cloc, lsel)
                        mb = _take16(valid, lsel) != 0
                        rb = jnp.full((16,), q * 16 + l, jnp.int32)
                        for j2 in range(hdim // 16):
                            cols = it16 + j2 * 16
                            av = plsc.load_gather(acc, [cb, cols])
                            gv = plsc.load_gather(grow, [rb, cols])
                            plsc.store_scatter(acc, [cb, cols],
                                               jnp.maximum(av, gv), mask=mb)
                    return carry3

                lax.fori_loop(0, 8, q_body, 0)
                return carry2

            lax.fori_loop(0, nblk, blk_body, 0)
            return carry

        lax.fori_loop(0, _NW, src_body, 0)

        @pl.when(w < _NW - 1)
        def _():
            pltpu.sync_copy(acc.at[pl.ds(0, _OWN)],
                            out_hbm.at[pl.ds(w * _OWN, _OWN)])

        @pl.when(w == _NW - 1)
        def _():
            pltpu.sync_copy(acc.at[pl.ds(0, npw_last)],
                            out_hbm.at[pl.ds((_NW - 1) * _OWN, npw_last)])

    return k(hact, parts, counts)


def kernel(x, edge_index, batch, params):
    n = x.shape[0]
    g = 64
    e = edge_index.shape[1]
    row, col = edge_index[0], edge_index[1]
    row2d = row.reshape(e // 128, 128)
    col2d = col.reshape(e // 128, 128)

    rows_each = ((n + _NS - 1) // _NS + 7) // 8 * 8
    zeros1d = jnp.zeros((rows_each,), jnp.float32)
    ones1d = jnp.ones((128,), jnp.float32)
    zeros2d = jnp.zeros((rows_each, x.shape[1] // 2), jnp.float32)
    zeros2p = jnp.zeros((64 // _NS, 256), jnp.float32)

    e_pad = -(-e // (_NS * 128)) * (_NS * 128)
    row2p = jnp.pad(row * 2, (0, e_pad - e),
                    constant_values=2 * n).reshape(e_pad // 128, 128)
    col2p = jnp.pad(col, (0, e_pad - e)).reshape(e_pad // 128, 128)

    degp = _sc_deg(col2d, zeros1d, ones1d, n).reshape(_NC, n)
    dinv = _dinv_from_partials(degp[0].reshape(n, 1), degp[1].reshape(n, 1))
    parts, counts = _sc_partition(row2d, col2d)

    h = x
    readout = None
    for i in range(1, 5):
        W, b = params[f'W{i}'], params[f'b{i}']
        lW, lb = params[f'lW{i}'], params[f'lb{i}']
        hws = _mm_scale(h, W, dinv)
        vals2p = jnp.pad(hws.reshape(2 * n, hws.shape[1] // 2),
                         ((0, 2), (0, 0)))
        part = _sc_scatter_rows(vals2p, row2p, col2p, zeros2d, n)
        h_act = _post_gcn(part, hws, dinv, b)
        h = _sc_seg_max(h_act, parts, counts, n)
        if readout is None:
            readout = jnp.zeros((n, lW.shape[1]), jnp.float32)
        readout = _readout_add(h, lW, lb, readout)

    n_pad = -(-n // 128) * 128
    readout_pad = jnp.pad(readout, ((0, n_pad - n), (0, 0)))
    batch_pad = jnp.pad(batch, (0, n_pad - n))
    gp = _sc_batch_sum(readout_pad, batch_pad, zeros2p, g)
    return _classifier(gp[0], gp[1], params)


# speedup vs baseline: 7.4593x; 1.3133x over previous
import functools

import jax
import jax.numpy as jnp
from jax import lax
from jax.experimental import pallas as pl
from jax.experimental.pallas import tpu as pltpu
from jax.experimental.pallas import tpu_sc as plsc

_NC, _NS = 2, 16
_NW = _NC * _NS
_SC_MESH = dict(mesh=plsc.VectorSubcoreMesh(core_axis_name="c",
                                            subcore_axis_name="s"))

_SELU_A = 1.6732632423543772
_SELU_S = 1.0507009873554805
_BN_S = 1.0 / (1.00001 ** 0.5)


def _mm_scale_body(h_ref, w_ref, dinv_ref, o_ref):
    hw = jnp.dot(h_ref[...], w_ref[...], preferred_element_type=jnp.float32)
    o_ref[...] = hw * dinv_ref[...]


def _mm_scale(h, W, dinv2d, block=1000):
    n, _ = h.shape
    o = W.shape[1]
    return pl.pallas_call(
        _mm_scale_body,
        grid=(n // block,),
        in_specs=[
            pl.BlockSpec((block, h.shape[1]), lambda i: (i, 0)),
            pl.BlockSpec(W.shape, lambda i: (0, 0)),
            pl.BlockSpec((block, 1), lambda i: (i, 0)),
        ],
        out_specs=pl.BlockSpec((block, o), lambda i: (i, 0)),
        out_shape=jax.ShapeDtypeStruct((n, o), jnp.float32),
    )(h, W, dinv2d)


def _dinv_body(p0_ref, p1_ref, o_ref):
    o_ref[...] = lax.rsqrt(1.0 + p0_ref[...] + p1_ref[...])


def _dinv_from_partials(p0, p1, block=1000):
    n = p0.shape[0]
    return pl.pallas_call(
        _dinv_body,
        grid=(n // block,),
        in_specs=[pl.BlockSpec((block, 1), lambda i: (i, 0))] * 2,
        out_specs=pl.BlockSpec((block, 1), lambda i: (i, 0)),
        out_shape=jax.ShapeDtypeStruct((n, 1), jnp.float32),
    )(p0, p1)


def _post_gcn_body(part_ref, hws_ref, dinv_ref, b_ref, o_ref):
    part = jnp.concatenate([part_ref[0], part_ref[1]], axis=-1)
    z = dinv_ref[...] * (part + hws_ref[...]) + b_ref[...]
    neg = _SELU_A * (jnp.exp(jnp.minimum(z, 0.0)) - 1.0)
    o_ref[...] = _SELU_S * jnp.where(z > 0, z, neg)


def _post_gcn(part3d, hws, dinv2d, b, block=1000):
    n, hdim = hws.shape
    half = hdim // 2
    return pl.pallas_call(
        _post_gcn_body,
        grid=(n // block,),
        in_specs=[
            pl.BlockSpec((2, block, half), lambda i: (0, i, 0)),
            pl.BlockSpec((block, hdim), lambda i: (i, 0)),
            pl.BlockSpec((block, 1), lambda i: (i, 0)),
            pl.BlockSpec((1, hdim), lambda i: (0, 0)),
        ],
        out_specs=pl.BlockSpec((block, hdim), lambda i: (i, 0)),
        out_shape=jax.ShapeDtypeStruct((n, hdim), jnp.float32),
    )(part3d, hws, dinv2d, b.reshape(1, hdim))


def _readout_body(h_ref, w_ref, b_ref, r_ref, o_ref):
    z = jnp.dot(h_ref[...], w_ref[...], preferred_element_type=jnp.float32)
    z = z + b_ref[...]
    z = z - jnp.max(z, axis=-1, keepdims=True)
    e = jnp.exp(z)
    o_ref[...] = r_ref[...] + e / jnp.sum(e, axis=-1, keepdims=True)


def _readout_add(h, lW, lb, r, block=1000):
    n, hdim = h.shape
    p = lW.shape[1]
    return pl.pallas_call(
        _readout_body,
        grid=(n // block,),
        in_specs=[
            pl.BlockSpec((block, hdim), lambda i: (i, 0)),
            pl.BlockSpec(lW.shape, lambda i: (0, 0)),
            pl.BlockSpec((1, p), lambda i: (0, 0)),
            pl.BlockSpec((block, p), lambda i: (i, 0)),
        ],
        out_specs=pl.BlockSpec((block, p), lambda i: (i, 0)),
        out_shape=jax.ShapeDtypeStruct((n, p), jnp.float32),
    )(h, lW, lb.reshape(1, p), r)


def _cls_body(ga_ref, gb_ref, w1, b1, g1, be1, w2, b2, g2, be2, w3, b3, g3, be3,
              w4, b4, o_ref):
    def lin(v, w, b):
        return jnp.dot(v, w[...], preferred_element_type=jnp.float32) + b[...]

    def bn(v, ga, be):
        return v * _BN_S * ga[...] + be[...]

    z = bn(jnp.maximum(lin(ga_ref[...] + gb_ref[...], w1, b1), 0.0), g1, be1)
    z = bn(jnp.maximum(lin(z, w2, b2), 0.0), g2, be2)
    z = bn(jnp.maximum(lin(z, w3, b3), 0.0), g3, be3)
    logits = lin(z, w4, b4)
    p = 1.0 / (1.0 + jnp.exp(-logits))
    p = p - jnp.max(p, axis=-1, keepdims=True)
    e = jnp.exp(p)
    o_ref[...] = e / jnp.sum(e, axis=-1, keepdims=True)


def _classifier(ga, gb, params):
    P = params
    args = [ga, gb]
    for names in (('cW1', 'cb1', 'g1', 'be1'),
                  ('cW2', 'cb2', 'g2', 'be2'),
                  ('cW3', 'cb3', 'g3', 'be3')):
        w, b, gm, be = (P[k] for k in names)
        args += [w, b.reshape(1, -1), gm.reshape(1, -1), be.reshape(1, -1)]
    args += [P['cW4'], P['cb4'].reshape(1, -1)]
    nout = P['cW4'].shape[1]
    specs = [pl.BlockSpec(a.shape, lambda i, _s=a.shape: (0,) * len(_s))
             for a in args]
    return pl.pallas_call(
        _cls_body,
        grid=(1,),
        in_specs=specs,
        out_specs=pl.BlockSpec((ga.shape[0], nout), lambda i: (0, 0)),
        out_shape=jax.ShapeDtypeStruct((ga.shape[0], nout), jnp.float32),
    )(*args)


def _zero_shared_rows(zeros_v, shared, sid, nrows_each, nrows_last):
    @pl.when(sid < _NS - 1)
    def _():
        pltpu.sync_copy(zeros_v, shared.at[pl.ds(sid * nrows_each,
                                                 nrows_each)])

    @pl.when(sid == _NS - 1)
    def _():
        pltpu.sync_copy(zeros_v.at[pl.ds(0, nrows_last)],
                        shared.at[pl.ds((_NS - 1) * nrows_each, nrows_last)])


def _sc_deg(col2d, zeros1d, ones1d, n):
    nchunk = col2d.shape[0]
    niter = (nchunk + _NW - 1) // _NW
    rows_each = ((n + _NS - 1) // _NS + 7) // 8 * 8
    rows_last = n - (_NS - 1) * rows_each

    @functools.partial(
        pl.kernel,
        out_type=jax.ShapeDtypeStruct((_NC * n,), jnp.float32),
        scratch_types=[
            pltpu.VMEM((128,), jnp.int32),
            pltpu.VMEM((128,), jnp.float32),
            pltpu.VMEM((rows_each,), jnp.float32),
            pltpu.VMEM_SHARED((n,), jnp.float32),
        ],
        **_SC_MESH,
    )
    def k(col_hbm, zeros_hbm, ones_hbm, out_hbm, idx_v, ones_v, zv, shared):
        cid = lax.axis_index("c")
        sid = lax.axis_index("s")
        wid = sid * _NC + cid
        pltpu.sync_copy(ones_hbm, ones_v)
        pltpu.sync_copy(zeros_hbm, zv)
        _zero_shared_rows(zv, shared, sid, rows_each, rows_last)
        plsc.subcore_barrier()

        def body(r, carry):
            ci = wid + _NW * r

            @pl.when(ci < nchunk)
            def _():
                pltpu.sync_copy(col_hbm.at[ci], idx_v)
                pltpu.sync_copy(ones_v, shared.at[idx_v], add=True)
            return carry

        lax.fori_loop(0, niter, body, 0)
        plsc.subcore_barrier()

        @pl.when(sid < _NS - 1)
        def _():
            pltpu.sync_copy(shared.at[pl.ds(sid * rows_each, rows_each)], zv)
            pltpu.sync_copy(
                zv, out_hbm.at[pl.ds(cid * n + sid * rows_each, rows_each)])

        @pl.when(sid == _NS - 1)
        def _():
            sl = pl.ds(0, rows_last)
            pltpu.sync_copy(
                shared.at[pl.ds((_NS - 1) * rows_each, rows_last)], zv.at[sl])
            pltpu.sync_copy(
                zv.at[sl],
                out_hbm.at[pl.ds(cid * n + (_NS - 1) * rows_each, rows_last)])

    return k(col2d, zeros1d, ones1d)


_NBUF = 4


def _sc_scatter_rows(vals2p, row2p, col2p, zeros2d, n):
    nchunk = row2p.shape[0]
    assert nchunk % _NS == 0
    niter = nchunk // _NS
    half = vals2p.shape[1]
    rows_each = ((n + _NS - 1) // _NS + 7) // 8 * 8
    rows_last = n - (_NS - 1) * rows_each

    @functools.partial(
        pl.kernel,
        out_type=jax.ShapeDtypeStruct((_NC, n, half), jnp.float32),
        scratch_types=[
            pltpu.VMEM((_NBUF, 128), jnp.int32),
            pltpu.VMEM((_NBUF, 128), jnp.int32),
            pltpu.VMEM((_NBUF, 128, half), jnp.float32),
            pltpu.VMEM((rows_each, half), jnp.float32),
            pltpu.VMEM_SHARED((n, half), jnp.float32),
            pltpu.SemaphoreType.DMA((_NBUF,)),
            pltpu.SemaphoreType.DMA((_NBUF,)),
            pltpu.SemaphoreType.DMA((_NBUF,)),
            pltpu.SemaphoreType.DMA((_NBUF,)),
        ],
        compiler_params=pltpu.CompilerParams(use_tc_tiling_on_sc=False),
        **_SC_MESH,
    )
    def k(vals_hbm, row_hbm, col_hbm, zeros_hbm, out_hbm,
          idxr, idxc, rows_v, zv, shared, semr, semc, semg, sems):
        cid = lax.axis_index("c")
        sid = lax.axis_index("s")
        pltpu.sync_copy(zeros_hbm, zv)
        _zero_shared_rows(zv, shared, sid, rows_each, rows_last)
        plsc.subcore_barrier()
        myvals = vals_hbm.at[pl.ds(cid, 2 * n + 1)]

        hidx, hg, hs = {}, {}, {}
        for t in range(niter + 2):
            if t < niter:
                b = t % _NBUF
                if t >= _NBUF:
                    hs[t - _NBUF].wait()
                ci = sid + _NS * t
                hidx[t] = (
                    pltpu.async_copy(row_hbm.at[ci], idxr.at[b], semr.at[b]),
                    pltpu.async_copy(col_hbm.at[ci], idxc.at[b], semc.at[b]))
            if 1 <= t < niter + 1:
                u, b = t - 1, (t - 1) % _NBUF
                hidx[u][0].wait()
                hg[u] = pltpu.async_copy(myvals.at[idxr.at[b]], rows_v.at[b],
                                         semg.at[b])
            if t >= 2:
                u, b = t - 2, (t - 2) % _NBUF
                hg[u].wait()
                hidx[u][1].wait()
                hs[u] = pltpu.async_copy(rows_v.at[b], shared.at[idxc.at[b]],
                                         sems.at[b], add=True)
        for t in range(max(0, niter - _NBUF), niter):
            hs[t].wait()
        plsc.subcore_barrier()

        @pl.when(sid < _NS - 1)
        def _():
            pltpu.sync_copy(shared.at[pl.ds(sid * rows_each, rows_each)], zv)
            pltpu.sync_copy(
                zv, out_hbm.at[cid, pl.ds(sid * rows_each, rows_each)])

        @pl.when(sid == _NS - 1)
        def _():
            sl = pl.ds(0, rows_last)
            pltpu.sync_copy(
                shared.at[pl.ds((_NS - 1) * rows_each, rows_last)], zv.at[sl])
            pltpu.sync_copy(
                zv.at[sl],
                out_hbm.at[cid, pl.ds((_NS - 1) * rows_each, rows_last)])

    return k(vals2p, row2p, col2p, zeros2d)


def _sc_batch_sum(vals, batch, zeros2d, g):
    n, p = vals.shape
    chunk = 128
    nchunk = n // chunk
    niter = (nchunk + _NW - 1) // _NW
    rows_each = g // _NS

    @functools.partial(
        pl.kernel,
        out_type=jax.ShapeDtypeStruct((_NC, g, p), jnp.float32),
        scratch_types=[
            pltpu.VMEM((chunk,), jnp.int32),
            pltpu.VMEM((chunk, p), jnp.float32),
            pltpu.VMEM((rows_each, p), jnp.float32),
            pltpu.VMEM_SHARED((g, p), jnp.float32),
        ],
        compiler_params=pltpu.CompilerParams(use_tc_tiling_on_sc=False),
        **_SC_MESH,
    )
    def k(vals_hbm, batch_hbm, zeros_hbm, out_hbm, idxb, val_v, zv, shared):
        cid = lax.axis_index("c")
        sid = lax.axis_index("s")
        wid = sid * _NC + cid
        pltpu.sync_copy(zeros_hbm, zv)
        pltpu.sync_copy(zv, shared.at[pl.ds(sid * rows_each, rows_each)])
        plsc.subcore_barrier()

        def body(r, carry):
            ci = wid + _NW * r

            @pl.when(ci < nchunk)
            def _():
                pltpu.sync_copy(batch_hbm.at[pl.ds(ci * chunk, chunk)], idxb)
                pltpu.sync_copy(vals_hbm.at[pl.ds(ci * chunk, chunk)], val_v)
                pltpu.sync_copy(val_v, shared.at[idxb], add=True)
            return carry

        lax.fori_loop(0, niter, body, 0)
        plsc.subcore_barrier()
        pltpu.sync_copy(shared.at[pl.ds(sid * rows_each, rows_each)], zv)
        pltpu.sync_copy(zv,
                        out_hbm.at[cid, pl.ds(sid * rows_each, rows_each)])

    return k(vals, batch, zeros2d)


_OWN = 313
_OWN_MUL, _OWN_SHIFT = 13401, 22
_REG = 10240
_PACK_SHIFT = 14


def _iota16():
    return lax.iota(jnp.int32, 16)


def _take16(x, idx):
    dnums = lax.GatherDimensionNumbers(offset_dims=(), collapsed_slice_dims=(0,),
                                       start_index_map=(0,))
    return lax.gather(x, idx[:, None], dnums, (1,),
                      mode=lax.GatherScatterMode.PROMISE_IN_BOUNDS)


def _sc_partition(row2d, col2d):
    nchunk = row2d.shape[0]
    niter = (nchunk + _NW - 1) // _NW

    @functools.partial(
        pl.kernel,
        out_type=(jax.ShapeDtypeStruct((_NW * _REG,), jnp.int32),
                  jax.ShapeDtypeStruct((_NW * _NW,), jnp.int32)),
        scratch_types=[
            pltpu.VMEM((128,), jnp.int32),
            pltpu.VMEM((128,), jnp.int32),
            pltpu.VMEM((_REG,), jnp.int32),
            pltpu.VMEM((_NW,), jnp.int32),
            pltpu.VMEM((_NW,), jnp.int32),
        ],
        compiler_params=pltpu.CompilerParams(use_tc_tiling_on_sc=False,
                                             needs_layout_passes=False),
        **_SC_MESH,
    )
    def k(row_hbm, col_hbm, parts_hbm, counts_hbm,
          cbuf, rbuf, outbuf, bins, wptr):
        cid = lax.axis_index("c")
        sid = lax.axis_index("s")
        w = sid * _NC + cid
        it16 = _iota16()
        zero16 = jnp.zeros((16,), jnp.int32)
        bins[pl.ds(0, 16)] = zero16
        bins[pl.ds(16, 16)] = zero16

        def owner_of(col16):
            return lax.shift_right_logical(col16 * _OWN_MUL, _OWN_SHIFT)

        def runs(os):
            prev = _take16(os, jnp.maximum(it16 - 1, 0))
            nxt = _take16(os, jnp.minimum(it16 + 1, 15))
            is_start = (it16 == 0) | (os != prev)
            is_end = (it16 == 15) | (os != nxt)
            run_base = plsc.cummax(jnp.where(is_start, it16, 0))
            rank = it16 - run_base
            return rank, is_end

        def count_body(r, carry):
            ci = w + _NW * r

            @pl.when(ci < nchunk)
            def _():
                pltpu.sync_copy(col_hbm.at[ci], cbuf)
                for q in range(8):
                    col16 = cbuf[pl.ds(q * 16, 16)]
                    os, _unused = plsc.sort_key_val(owner_of(col16), it16)
                    rank, is_end = runs(os)
                    plsc.addupdate_scatter(bins, [os], rank + 1, mask=is_end)
            return carry

        lax.fori_loop(0, niter, count_body, 0)
        pltpu.sync_copy(bins, counts_hbm.at[pl.ds(w * _NW, _NW)])

        b0 = bins[pl.ds(0, 16)]
        b1 = bins[pl.ds(16, 16)]
        c0 = plsc.cumsum(b0)
        t0 = jnp.max(c0)
        wptr[pl.ds(0, 16)] = c0 - b0
        wptr[pl.ds(16, 16)] = plsc.cumsum(b1) - b1 + t0

        def place_body(r, carry):
            ci = w + _NW * r

            @pl.when(ci < nchunk)
            def _():
                pltpu.sync_copy(col_hbm.at[ci], cbuf)
                pltpu.sync_copy(row_hbm.at[ci], rbuf)
                for q in range(8):
                    col16 = cbuf[pl.ds(q * 16, 16)]
                    row16 = rbuf[pl.ds(q * 16, 16)]
                    os, perm = plsc.sort_key_val(owner_of(col16), it16)
                    row_s = _take16(row16, perm)
                    col_s = _take16(col16, perm)
                    rank, is_end = runs(os)
                    base = plsc.load_gather(wptr, [os])
                    packed = row_s | lax.shift_left(col_s - os * _OWN,
                                                    _PACK_SHIFT)
                    plsc.store_scatter(outbuf, [base + rank], packed)
                    plsc.addupdate_scatter(wptr, [os], rank + 1, mask=is_end)
            return carry

        lax.fori_loop(0, niter, place_body, 0)

        nfull = nchunk // _NW

        @pl.when(w < nchunk - nfull * _NW)
        def _():
            for q in range((_REG - (nfull + 1) * 128) // 16):
                outbuf[pl.ds((nfull + 1) * 128 + q * 16, 16)] = zero16

        @pl.when(w >= nchunk - nfull * _NW)
        def _():
            for q in range((_REG - nfull * 128) // 16):
                outbuf[pl.ds(nfull * 128 + q * 16, 16)] = zero16

        pltpu.sync_copy(outbuf, parts_hbm.at[pl.ds(w * _REG, _REG)])

    return k(row2d, col2d)


def _sc_seg_max(hact, parts, counts, n):
    hdim = hact.shape[1]
    npw_last = n - (_NW - 1) * _OWN
    tmax = parts.shape[0] // 128 + 2 * _NW + 8

    @functools.partial(
        pl.kernel,
        out_type=jax.ShapeDtypeStruct((n, hdim), jnp.float32),
        scratch_types=[
            pltpu.VMEM((320, hdim), jnp.float32),
            pltpu.VMEM((_NW * _NW,), jnp.int32),
            pltpu.VMEM((3, 128), jnp.int32),
            pltpu.VMEM((3, 128), jnp.int32),
            pltpu.VMEM((3, 128), jnp.int32),
            pltpu.VMEM((3, 128, hdim), jnp.float32),
            pltpu.VMEM((tmax,), jnp.int32),
            pltpu.VMEM((tmax,), jnp.int32),
            pltpu.VMEM((tmax,), jnp.int32),
            pltpu.SemaphoreType.DMA((3,)),
            pltpu.SemaphoreType.DMA((3,)),
        ],
        compiler_params=pltpu.CompilerParams(use_tc_tiling_on_sc=False,
                                             needs_layout_passes=False),
        **_SC_MESH,
    )
    def k(hact_hbm, parts_hbm, counts_hbm, out_hbm,
          acc, cntm, pbuf, ridx, clocb, grow, boff_v, brel_v, bcnt_v,
          semp, semg):
        cid = lax.axis_index("c")
        sid = lax.axis_index("s")
        w = sid * _NC + cid
        it16 = _iota16()
        z16 = jnp.zeros((16,), jnp.int32)
        lane0 = it16 == 0
        pltpu.sync_copy(counts_hbm, cntm)

        @pl.when(w < _NW - 1)
        def _():
            pltpu.sync_copy(hact_hbm.at[pl.ds(w * _OWN, _OWN)],
                            acc.at[pl.ds(0, _OWN)])

        @pl.when(w == _NW - 1)
        def _():
            pltpu.sync_copy(hact_hbm.at[pl.ds((_NW - 1) * _OWN, npw_last)],
                            acc.at[pl.ds(0, npw_last)])

        def sget(buf, i):
            return jnp.max(plsc.load_gather(buf, [z16 + i]))

        def sput(buf, i, val):
            plsc.store_scatter(buf, [z16 + i], z16 + val, mask=lane0)

        def meta_body(v, t):
            vbase = pl.multiple_of(v * _NW, _NW)
            a = cntm[pl.ds(vbase, 16)]
            b = cntm[pl.ds(vbase + 16, 16)]
            off = (jnp.sum(jnp.where(it16 < w, a, 0))
                   + jnp.sum(jnp.where(it16 < w - 16, b, 0)))
            cnt = (jnp.sum(jnp.where(it16 == w, a, 0))
                   + jnp.sum(jnp.where(it16 == w - 16, b, 0)))
            base = v * _REG + off
            st = lax.shift_left(lax.shift_right_logical(base, 3), 3)
            nblk = (base + cnt - st + 127) // 128
            head = base - st

            def blk_meta(j, t2):
                sput(boff_v, t2, st + j * 128)
                sput(brel_v, t2, j * 128 - head)
                sput(bcnt_v, t2, cnt)
                return t2 + 1

            return lax.fori_loop(0, nblk, blk_meta, t)

        tcount = lax.fori_loop(0, _NW, meta_body, 0)

        def issue_parts(g):
            b = lax.rem(g, 3)
            o8 = pl.multiple_of(sget(boff_v, g), 8)
            pltpu.async_copy(parts_hbm.at[pl.ds(o8, 128)], pbuf.at[b],
                             semp.at[b])

        def wait_parts(b):
            pltpu.make_async_copy(parts_hbm.at[pl.ds(0, 128)], pbuf.at[b],
                                  semp.at[b]).wait()

        def stage_gather(g):
            b = lax.rem(g, 3)
            wait_parts(b)
            bv = z16 + b
            for q in range(8):
                pv = plsc.load_gather(pbuf, [bv, q * 16 + it16])
                ridx[b, pl.ds(q * 16, 16)] = pv & ((1 << _PACK_SHIFT) - 1)
                clocb[b, pl.ds(q * 16, 16)] = lax.shift_right_logical(
                    pv, _PACK_SHIFT)
            pltpu.async_copy(hact_hbm.at[ridx.at[b]], grow.at[b], semg.at[b])

        def process(g):
            b = lax.rem(g, 3)
            bv = z16 + b
            pltpu.make_async_copy(hact_hbm.at[pl.ds(0, 128)], grow.at[b],
                                  semg.at[b]).wait()
            relb = sget(brel_v, g)
            cnt = sget(bcnt_v, g)

            def q_body(q, carry3):
                cloc = plsc.load_gather(clocb, [bv, q * 16 + it16])
                rel = relb + q * 16 + it16
                valid = ((rel >= 0) & (rel < cnt)).astype(jnp.int32)
                for l in range(16):
                    lsel = jnp.full((16,), l, jnp.int32)
                    cb = _take16(cloc, lsel)
                    mb = _take16(valid, lsel) != 0
                    rb = jnp.full((16,), q * 16 + l, jnp.int32)
                    for j2 in range(hdim // 16):
                        cols = it16 + j2 * 16
                        av = plsc.load_gather(acc, [cb, cols])
                        gv = plsc.load_gather(grow, [bv, rb, cols])
                        plsc.store_scatter(acc, [cb, cols],
                                           jnp.maximum(av, gv), mask=mb)
                return carry3

            lax.fori_loop(0, 8, q_body, 0)

        @pl.when(tcount >= 1)
        def _():
            issue_parts(0)

            @pl.when(tcount >= 2)
            def _():
                issue_parts(1)
            stage_gather(0)

            def pipe_body(g, carry):
                @pl.when(g + 2 < tcount)
                def _():
                    issue_parts(g + 2)

                @pl.when(g + 1 < tcount)
                def _():
                    stage_gather(g + 1)
                process(g)
                return carry

            lax.fori_loop(0, tcount, pipe_body, 0)

        @pl.when(w < _NW - 1)
        def _():
            pltpu.sync_copy(acc.at[pl.ds(0, _OWN)],
                            out_hbm.at[pl.ds(w * _OWN, _OWN)])

        @pl.when(w == _NW - 1)
        def _():
            pltpu.sync_copy(acc.at[pl.ds(0, npw_last)],
                            out_hbm.at[pl.ds((_NW - 1) * _OWN, npw_last)])

    return k(hact, parts, counts)


def kernel(x, edge_index, batch, params):
    n = x.shape[0]
    g = 64
    e = edge_index.shape[1]
    row, col = edge_index[0], edge_index[1]
    row2d = row.reshape(e // 128, 128)
    col2d = col.reshape(e // 128, 128)

    rows_each = ((n + _NS - 1) // _NS + 7) // 8 * 8
    zeros1d = jnp.zeros((rows_each,), jnp.float32)
    ones1d = jnp.ones((128,), jnp.float32)
    zeros2d = jnp.zeros((rows_each, x.shape[1] // 2), jnp.float32)
    zeros2p = jnp.zeros((64 // _NS, 256), jnp.float32)

    e_pad = -(-e // (_NS * 128)) * (_NS * 128)
    row2p = jnp.pad(row * 2, (0, e_pad - e),
                    constant_values=2 * n).reshape(e_pad // 128, 128)
    col2p = jnp.pad(col, (0, e_pad - e)).reshape(e_pad // 128, 128)

    degp = _sc_deg(col2d, zeros1d, ones1d, n).reshape(_NC, n)
    dinv = _dinv_from_partials(degp[0].reshape(n, 1), degp[1].reshape(n, 1))
    parts, counts = _sc_partition(row2d, col2d)

    h = x
    readout = None
    for i in range(1, 5):
        W, b = params[f'W{i}'], params[f'b{i}']
        lW, lb = params[f'lW{i}'], params[f'lb{i}']
        hws = _mm_scale(h, W, dinv)
        vals2p = jnp.pad(hws.reshape(2 * n, hws.shape[1] // 2),
                         ((0, 2), (0, 0)))
        part = _sc_scatter_rows(vals2p, row2p, col2p, zeros2d, n)
        h_act = _post_gcn(part, hws, dinv, b)
        h = _sc_seg_max(h_act, parts, counts, n)
        if readout is None:
            readout = jnp.zeros((n, lW.shape[1]), jnp.float32)
        readout = _readout_add(h, lW, lb, readout)

    n_pad = -(-n // 128) * 128
    readout_pad = jnp.pad(readout, ((0, n_pad - n), (0, 0)))
    batch_pad = jnp.pad(batch, (0, n_pad - n))
    gp = _sc_batch_sum(readout_pad, batch_pad, zeros2p, g)
    return _classifier(gp[0], gp[1], params)
